# Initial kernel scaffold; baseline (speedup 1.0000x reference)
import jax
import jax.numpy as jnp
from jax.experimental import pallas as pl


def kernel(h_v, edge_index, fc_W, fc_b, pi_w):
    raise NotImplementedError("write your pallas kernel here")


# speedup vs baseline: 3.8330x; 3.8330x over previous
import functools

import jax
import jax.numpy as jnp
from jax import lax
from jax.experimental import pallas as pl
from jax.experimental.pallas import tpu as pltpu
from jax.experimental.pallas import tpu_sc as plsc

N = 10000
E = 320000
DIM = 128
HEADS = 2
DH = DIM * HEADS

NP = 10240
SL = NP // 16
NW = 32
C = 80
EW2 = E // 16
NCH2 = EW2 // C
EW1 = E // NW
NCH1 = EW1 // C

_LANE = None


def _gat16(x, idx):
    dn = lax.GatherDimensionNumbers(
        offset_dims=(), collapsed_slice_dims=(0,), start_index_map=(0,))
    return lax.gather(x, idx[:, None], dimension_numbers=dn,
                      slice_sizes=(1,),
                      mode=lax.GatherScatterMode.PROMISE_IN_BOUNDS)


def _seg_accum(ref, keys, vals, op):
    sd, sv = plsc.sort_key_val(keys, vals)
    iota = lax.iota(jnp.int32, 16)
    acc = sv
    for step in (1, 2, 4, 8):
        srcl = iota - step
        valid = srcl >= 0
        srcc = jnp.maximum(srcl, 0)
        pk = _gat16(sd, srcc)
        pv = _gat16(acc, srcc)
        same = jnp.logical_and(valid, pk == sd)
        comb = jnp.maximum(acc, pv) if op == "max" else acc + pv
        acc = jnp.where(same, comb, acc)
    nxt = _gat16(sd, jnp.minimum(iota + 1, 15))
    is_last = jnp.logical_or(iota == 15, nxt != sd)
    cur = plsc.load_gather(ref, [sd])
    newv = jnp.maximum(cur, acc) if op == "max" else cur + acc
    plsc.store_scatter(ref, [sd], newv, mask=is_last)


def _k1_body(h_ref, w_ref, b_ref, piw_ref, ft_ref, g_ref):
    ft = jnp.dot(h_ref[...], w_ref[...],
                 preferred_element_type=jnp.float32) + b_ref[...]
    ft_ref[...] = ft
    g_ref[...] = ft * piw_ref[...]


def _k1(hp, fc_W, fc_b_row, piw_row):
    blk = 256
    grid = NP // blk
    return pl.pallas_call(
        _k1_body,
        grid=(grid,),
        in_specs=[
            pl.BlockSpec((blk, DIM), lambda i: (i, 0)),
            pl.BlockSpec((DIM, DH), lambda i: (0, 0)),
            pl.BlockSpec((1, DH), lambda i: (0, 0)),
            pl.BlockSpec((1, DH), lambda i: (0, 0)),
        ],
        out_specs=[
            pl.BlockSpec((blk, DH), lambda i: (i, 0)),
            pl.BlockSpec((blk, DH), lambda i: (i, 0)),
        ],
        out_shape=[
            jax.ShapeDtypeStruct((NP, DH), jnp.float32),
            jax.ShapeDtypeStruct((NP, DH), jnp.float32),
        ],
    )(hp, fc_W, fc_b_row, piw_row)


def _k2_body(g_hbm, ft_hbm, src_hbm, dst_hbm, e_hbm, emax_hbm,
             idx_s, idx_d, gbuf, fbuf, ebuf, emax_loc, dots, sem1, sem2):
    cid = lax.axis_index("c")
    sid = lax.axis_index("s")
    wid = cid * 16 + sid
    lane = lax.iota(jnp.int32, 16)
    neg_inf = jnp.full((16,), -jnp.inf, jnp.float32)

    def init_emax(i, _):
        emax_loc[pl.ds(i * 16, 16)] = neg_inf
        return 0
    lax.fori_loop(0, NP // 16, init_emax, 0)

    base = wid * EW1

    def chunk(j, _):
        bj = base + j * C
        pltpu.sync_copy(src_hbm.at[pl.ds(bj, C)], idx_s)
        pltpu.sync_copy(dst_hbm.at[pl.ds(bj, C)], idx_d)
        cp1 = pltpu.async_copy(g_hbm.at[idx_s], gbuf, sem1)
        cp2 = pltpu.async_copy(ft_hbm.at[idx_d], fbuf, sem2)
        cp1.wait()
        cp2.wait()

        def per_edge(i, _):
            acc = jnp.zeros((16,), jnp.float32)
            for k in range(DH // 16):
                acc = acc + (gbuf[i, pl.ds(16 * k, 16)]
                             * fbuf[i, pl.ds(16 * k, 16)])
            dots[pl.ds(16 * i, 16)] = jnp.cumsum(acc)
            return 0
        lax.fori_loop(0, C, per_edge, 0)

        for t in range(C // 16):
            idx15 = (lane + 16 * t) * 16 + 15
            d = plsc.load_gather(dots, [idx15])
            e16 = jnp.where(d > 0, d, 0.2 * d)
            ebuf[pl.ds(16 * t, 16)] = e16
            d16 = idx_d[pl.ds(16 * t, 16)]
            _seg_accum(emax_loc, d16, e16, "max")
        pltpu.sync_copy(ebuf, e_hbm.at[pl.ds(bj, C)])
        return 0
    lax.fori_loop(0, NCH1, chunk, 0)
    pltpu.sync_copy(emax_loc, emax_hbm.at[wid])


def _k2(g, ft, src, dst):
    mesh = plsc.VectorSubcoreMesh(core_axis_name="c", subcore_axis_name="s")
    kfn = pl.kernel(
        _k2_body,
        mesh=mesh,
        compiler_params=pltpu.CompilerParams(use_tc_tiling_on_sc=False,
                                             needs_layout_passes=False),
        out_type=[
            jax.ShapeDtypeStruct((E,), jnp.float32),
            jax.ShapeDtypeStruct((NW, NP), jnp.float32),
        ],
        scratch_types=[
            pltpu.VMEM((C,), jnp.int32),
            pltpu.VMEM((C,), jnp.int32),
            pltpu.VMEM((C, DH), jnp.float32),
            pltpu.VMEM((C, DH), jnp.float32),
            pltpu.VMEM((C,), jnp.float32),
            pltpu.VMEM((NP,), jnp.float32),
            pltpu.VMEM((C * 16,), jnp.float32),
            pltpu.SemaphoreType.DMA,
            pltpu.SemaphoreType.DMA,
        ],
    )
    return kfn(g, ft, src, dst)


def _k3_body(ft128_hbm, src_hbm, dst_hbm, e_hbm, emaxp_hbm,
             rst_hbm, ex_hbm, dpart_hbm,
             emax_loc, denom_loc, rowbuf, dstc, sidx, ebuf, abuf,
             tmpa, tmpb, sem,
             emax_sh, acc_sh):
    cid = lax.axis_index("c")
    sid = lax.axis_index("s")
    ebase = sid * EW2
    s0 = sid * SL
    z16 = jnp.zeros((16,), jnp.float32)
    neg_inf = jnp.full((16,), -jnp.inf, jnp.float32)

    def zrow(i, _):
        for k in range(DIM // 16):
            rowbuf[i, pl.ds(16 * k, 16)] = z16
        return 0
    lax.fori_loop(0, C, zrow, 0)
    for m in range(SL // C):
        pltpu.sync_copy(rowbuf, acc_sh.at[pl.ds(s0 + m * C, C)])

    def init_a(t, _):
        tmpa[pl.ds(16 * t, 16)] = neg_inf
        return 0
    lax.fori_loop(0, SL // 16, init_a, 0)

    def rmax(r, _):
        pltpu.sync_copy(emaxp_hbm.at[r, pl.ds(s0, SL)], tmpb)

        def mstep(t, _):
            tmpa[pl.ds(t * 16, 16)] = jnp.maximum(tmpa[pl.ds(t * 16, 16)],
                                                  tmpb[pl.ds(t * 16, 16)])
            return 0
        lax.fori_loop(0, SL // 16, mstep, 0)
        return 0
    lax.fori_loop(0, NW, rmax, 0)
    pltpu.sync_copy(tmpa, emax_sh.at[pl.ds(s0, SL)])
    plsc.subcore_barrier()
    pltpu.sync_copy(emax_sh, emax_loc)

    def init_d(i, _):
        denom_loc[pl.ds(i * 16, 16)] = z16
        return 0
    lax.fori_loop(0, NP // 16, init_d, 0)

    def chB(j, _):
        bj = ebase + j * C
        pltpu.sync_copy(e_hbm.at[pl.ds(bj, C)], ebuf)
        pltpu.sync_copy(dst_hbm.at[pl.ds(bj, C)], dstc)
        for t in range(C // 16):
            d16 = dstc[pl.ds(16 * t, 16)]
            em = plsc.load_gather(emax_loc, [d16])
            ex = jnp.exp(ebuf[pl.ds(16 * t, 16)] - em)
            ebuf[pl.ds(16 * t, 16)] = ex
            _seg_accum(denom_loc, d16, ex, "add")
        pltpu.sync_copy(ebuf, ex_hbm.at[cid, pl.ds(bj, C)])
        return 0
    lax.fori_loop(0, NCH2, chB, 0)

    pltpu.sync_copy(denom_loc, dpart_hbm.at[cid, sid])
    plsc.subcore_barrier()

    def init_s(t, _):
        tmpa[pl.ds(16 * t, 16)] = z16
        return 0
    lax.fori_loop(0, SL // 16, init_s, 0)

    def rsum(r, _):
        pltpu.sync_copy(dpart_hbm.at[cid, r, pl.ds(s0, SL)], tmpb)

        def astep(t, _):
            tmpa[pl.ds(t * 16, 16)] = (tmpa[pl.ds(t * 16, 16)]
                                       + tmpb[pl.ds(t * 16, 16)])
            return 0
        lax.fori_loop(0, SL // 16, astep, 0)
        return 0
    lax.fori_loop(0, 16, rsum, 0)

    def inv_step(t, _):
        v = tmpa[pl.ds(t * 16, 16)]
        tmpa[pl.ds(t * 16, 16)] = 1.0 / (v + 1e-9)
        return 0
    lax.fori_loop(0, SL // 16, inv_step, 0)
    pltpu.sync_copy(tmpa, emax_sh.at[pl.ds(s0, SL)])
    plsc.subcore_barrier()
    pltpu.sync_copy(emax_sh, emax_loc)

    def chD(j, _):
        bj = ebase + j * C
        pltpu.sync_copy(src_hbm.at[pl.ds(bj, C)], sidx)
        pltpu.sync_copy(dst_hbm.at[pl.ds(bj, C)], dstc)
        pltpu.sync_copy(ex_hbm.at[cid, pl.ds(bj, C)], ebuf)
        for t in range(C // 16):
            sidx[pl.ds(16 * t, 16)] = sidx[pl.ds(16 * t, 16)] * 2 + cid
        pltpu.async_copy(ft128_hbm.at[sidx], rowbuf, sem).wait()
        for t in range(C // 16):
            d16 = dstc[pl.ds(16 * t, 16)]
            iv = plsc.load_gather(emax_loc, [d16])
            abuf[pl.ds(16 * t, 16)] = ebuf[pl.ds(16 * t, 16)] * iv
        def rowscale(i, _):
            a = plsc.load_gather(abuf, [jnp.full((16,), i, jnp.int32)])
            for k in range(DIM // 16):
                rowbuf[i, pl.ds(16 * k, 16)] = rowbuf[i, pl.ds(16 * k, 16)] * a
            return 0
        lax.fori_loop(0, C, rowscale, 0)
        pltpu.sync_copy(rowbuf, acc_sh.at[dstc], add=True)
        return 0
    lax.fori_loop(0, NCH2, chD, 0)

    plsc.subcore_barrier()
    pltpu.sync_copy(acc_sh.at[pl.ds(s0, SL)], rst_hbm.at[cid, pl.ds(s0, SL)])


def _k3(ft128, src, dst, e, emax_part):
    mesh = plsc.VectorSubcoreMesh(core_axis_name="c", subcore_axis_name="s")
    kfn = pl.kernel(
        _k3_body,
        mesh=mesh,
        compiler_params=pltpu.CompilerParams(use_tc_tiling_on_sc=False,
                                             needs_layout_passes=False),
        out_type=[
            jax.ShapeDtypeStruct((HEADS, NP, DIM), jnp.float32),
            jax.ShapeDtypeStruct((HEADS, E), jnp.float32),
            jax.ShapeDtypeStruct((HEADS, 16, NP), jnp.float32),
        ],
        scratch_types=[
            pltpu.VMEM((NP,), jnp.float32),
            pltpu.VMEM((NP,), jnp.float32),
            pltpu.VMEM((C, DIM), jnp.float32),
            pltpu.VMEM((C,), jnp.int32),
            pltpu.VMEM((C,), jnp.int32),
            pltpu.VMEM((C,), jnp.float32),
            pltpu.VMEM((C,), jnp.float32),
            pltpu.VMEM((SL,), jnp.float32),
            pltpu.VMEM((SL,), jnp.float32),
            pltpu.SemaphoreType.DMA,
            pltpu.VMEM_SHARED((NP,), jnp.float32),
            pltpu.VMEM_SHARED((NP, DIM), jnp.float32),
        ],
    )
    rst, _, _ = kfn(ft128, src, dst, e, emax_part)
    return rst


def _k4_body(rst_ref, out_ref):
    out_ref[...] = jnp.max(rst_ref[...], axis=0)


def _k4(rst):
    blk = 400
    grid = N // blk
    return pl.pallas_call(
        _k4_body,
        grid=(grid,),
        in_specs=[pl.BlockSpec((HEADS, blk, DIM), lambda i: (0, i, 0))],
        out_specs=pl.BlockSpec((blk, DIM), lambda i: (i, 0)),
        out_shape=jax.ShapeDtypeStruct((N, DIM), jnp.float32),
    )(rst)


def kernel(h_v, edge_index, fc_W, fc_b, pi_w):
    hp = jnp.pad(h_v, ((0, NP - N), (0, 0)))
    ft, g = _k1(hp, fc_W, fc_b.reshape(1, DH), pi_w.reshape(1, DH))
    src = edge_index[0].astype(jnp.int32)
    dst = edge_index[1].astype(jnp.int32)
    e, emax_part = _k2(g, ft, src, dst)
    ft128 = ft.reshape(HEADS * NP, DIM)
    rst = _k3(ft128, src, dst, e, emax_part)
    return _k4(rst)
